# baseline (speedup 1.0000x reference)
import functools

import jax
import jax.numpy as jnp
from jax import lax
from jax.experimental import pallas as pl
from jax.experimental.pallas import tpu as pltpu
from jax.experimental.pallas import tpu_sc as plsc

N = 10000
E = 320000
F = 128
NL = 2
NS_STEPS = 4

NC = 2
NSUB = 16
K = 80
EPT = E // NSUB
NCHUNK = EPT // K
NP = 10240
ROWS_PT = NP // NSUB

_mesh = plsc.VectorSubcoreMesh(
    core_axis_name="c", subcore_axis_name="s", num_cores=NC, num_subcores=NSUB)


@functools.partial(
    pl.kernel,
    out_type=pltpu.HBM((2 * NP, F), jnp.float32),
    mesh=_mesh,
    scratch_types=[
        pltpu.VMEM((EPT,), jnp.int32),
        pltpu.VMEM((2, K), jnp.int32),
        pltpu.VMEM((2 * K, F), jnp.float32),
        pltpu.VMEM((K,), jnp.int32),
        pltpu.VMEM_SHARED((NP, F), jnp.float32),
        pltpu.SemaphoreType.DMA((2,)),
        pltpu.SemaphoreType.DMA((2,)),
    ],
)
def _agg(tbl_hbm, src_hbm, dst_hbm, out,
         src_v, dsts, rows, idx_z, acc, gsems, ssems):
    c = lax.axis_index("c")
    s = lax.axis_index("s")

    e0 = s * EPT
    pltpu.sync_copy(src_hbm.at[pl.ds(e0, EPT)], src_v)
    coff = c * N

    def shift(i, carry):
        src_v[pl.ds(16 * i, 16)] = src_v[pl.ds(16 * i, 16)] + coff
        return carry

    lax.fori_loop(0, EPT // 16, shift, 0)

    zero16 = jnp.zeros((16,), jnp.float32)

    def zrow(r, carry):
        for k in range(F // 16):
            rows[r, pl.ds(16 * k, 16)] = zero16
        return carry

    lax.fori_loop(0, K, zrow, 0)
    r0 = s * ROWS_PT
    iota16 = lax.iota(jnp.int32, 16)
    for q in range(ROWS_PT // K):
        for k in range(K // 16):
            idx_z[pl.ds(16 * k, 16)] = iota16 + (r0 + q * K + 16 * k)
        pltpu.sync_copy(rows.at[pl.ds(0, K)], acc.at[idx_z])
    plsc.subcore_barrier()

    R = 2

    def issue_gather(j, b):
        pltpu.async_copy(tbl_hbm.at[src_v.at[pl.ds(j * K, K)]],
                         rows.at[pl.ds(b * K, K)], gsems.at[b])

    def wait_gather(b):
        pltpu.make_async_copy(tbl_hbm.at[src_v.at[pl.ds(0, K)]],
                              rows.at[pl.ds(b * K, K)], gsems.at[b]).wait()

    def issue_scatter(b):
        pltpu.async_copy(rows.at[pl.ds(b * K, K)], acc.at[dsts.at[b]],
                         ssems.at[b], add=True)

    def wait_scatter(b):
        pltpu.make_async_copy(rows.at[pl.ds(b * K, K)], acc.at[dsts.at[b]],
                              ssems.at[b]).wait()

    def load_dst(j, b):
        pltpu.sync_copy(dst_hbm.at[pl.ds(e0 + j * K, K)], dsts.at[b])

    def prime(b, carry):
        load_dst(b, b)
        issue_gather(b, b)
        return carry

    lax.fori_loop(0, R, prime, 0)

    def body(j, carry):
        b = lax.rem(j, R)
        wait_gather(b)
        issue_scatter(b)

        @pl.when(j + R < NCHUNK)
        def _():
            wait_scatter(b)
            load_dst(j + R, b)
            issue_gather(j + R, b)

        return carry

    lax.fori_loop(0, NCHUNK, body, 0)

    def drain(b, carry):
        wait_scatter(b)
        return carry

    lax.fori_loop(0, R, drain, 0)
    plsc.subcore_barrier()

    ob = c * NP + r0
    for q in range(ROWS_PT // K):
        pltpu.sync_copy(acc.at[pl.ds(r0 + q * K, K)], rows.at[pl.ds(0, K)])
        pltpu.sync_copy(rows.at[pl.ds(0, K)], out.at[pl.ds(ob + q * K, K)])


_RB = 1000


def _cell_body(ax_ref, ah_ref, deg_ref, h_ref, w_ref, b_ref, out_ref):
    di = 1.0 / jnp.maximum(deg_ref[...], 1.0)
    m = jnp.concatenate([ax_ref[...] * di, ah_ref[...] * di], axis=1)
    pre = jnp.dot(m, w_ref[...], preferred_element_type=jnp.float32) + b_ref[...]
    u = jax.nn.sigmoid(pre[:, :F])
    cand = jnp.tanh(pre[:, F:])
    h = h_ref[...]
    out_ref[...] = u * h + (1.0 - u) * cand


_cell = pl.pallas_call(
    _cell_body,
    grid=(N // _RB,),
    in_specs=[
        pl.BlockSpec((_RB, F), lambda i: (i, 0)),
        pl.BlockSpec((_RB, F), lambda i: (i, 0)),
        pl.BlockSpec((_RB, 1), lambda i: (i, 0)),
        pl.BlockSpec((_RB, F), lambda i: (i, 0)),
        pl.BlockSpec((2 * F, 2 * F), lambda i: (0, 0)),
        pl.BlockSpec((1, 2 * F), lambda i: (0, 0)),
    ],
    out_specs=pl.BlockSpec((_RB, F), lambda i: (i, 0)),
    out_shape=jax.ShapeDtypeStruct((N, F), jnp.float32),
)


def _proj_body(y_ref, w_ref, b_ref, out_ref):
    out_ref[...] = (
        jnp.dot(y_ref[...], w_ref[...], preferred_element_type=jnp.float32)
        + b_ref[...])


_proj = pl.pallas_call(
    _proj_body,
    grid=(NS_STEPS * N // _RB,),
    in_specs=[
        pl.BlockSpec((_RB, F), lambda i: (i, 0)),
        pl.BlockSpec((F, F), lambda i: (0, 0)),
        pl.BlockSpec((1, F), lambda i: (0, 0)),
    ],
    out_specs=pl.BlockSpec((_RB, F), lambda i: (i, 0)),
    out_shape=jax.ShapeDtypeStruct((NS_STEPS * N, F), jnp.float32),
)


def kernel(x, hidden_states, W, B, Wo, bo, edge_index):
    src = edge_index[0]
    dst = edge_index[1]

    ones_tbl = jnp.ones((2 * N, F), jnp.float32)
    deg = _agg(ones_tbl, src, dst)[:N, :1]

    wcat, bcat = [], []
    for j in range(NL):
        wj = jnp.concatenate([
            jnp.concatenate([W[j, 2], W[j, 4]], axis=1),
            jnp.concatenate([W[j, 3], W[j, 5]], axis=1)], axis=0)
        bj = jnp.concatenate([B[j, 2] + B[j, 3], B[j, 4] + B[j, 5]])[None, :]
        wcat.append(wj)
        bcat.append(bj)

    hid = [hidden_states[j] for j in range(NL)]
    outs = []
    for i in range(NS_STEPS):
        inp = x[i]
        for j in range(NL):
            tbl = jnp.concatenate([inp, hid[j]], axis=0)
            agg = _agg(tbl, src, dst)
            inp = _cell(agg[:N], agg[NP:NP + N], deg, hid[j],
                        wcat[j], bcat[j])
            hid[j] = inp
        outs.append(inp)

    y = jnp.stack(outs).reshape(NS_STEPS * N, F)
    out = _proj(y, Wo, bo[None, :]).reshape(NS_STEPS, N, F)
    return (out, jnp.stack(hid))


# speedup vs baseline: 1.2074x; 1.2074x over previous
import functools

import jax
import jax.numpy as jnp
from jax import lax
from jax.experimental import pallas as pl
from jax.experimental.pallas import tpu as pltpu
from jax.experimental.pallas import tpu_sc as plsc

N = 10000
E = 320000
F = 128
NL = 2
NS_STEPS = 4

NC = 2
NSUB = 16
K = 80
EPT = E // NSUB
NCHUNK = EPT // K
NP = 10240
ROWS_PT = NP // NSUB

_mesh = plsc.VectorSubcoreMesh(
    core_axis_name="c", subcore_axis_name="s", num_cores=NC, num_subcores=NSUB)


@functools.partial(
    pl.kernel,
    out_type=pltpu.HBM((2 * NP, F), jnp.float32),
    mesh=_mesh,
    scratch_types=[
        pltpu.VMEM((EPT,), jnp.int32),
        pltpu.VMEM((2, K), jnp.int32),
        pltpu.VMEM((2 * K, F), jnp.float32),
        pltpu.VMEM((K,), jnp.int32),
        pltpu.VMEM_SHARED((NP, F), jnp.float32),
        pltpu.SemaphoreType.DMA((2,)),
        pltpu.SemaphoreType.DMA((2,)),
        pltpu.SemaphoreType.DMA((2,)),
    ],
)
def _agg(tbl_hbm, src_hbm, dst_hbm, out,
         src_v, dsts, rows, idx_z, acc, gsems, ssems, dsems):
    c = lax.axis_index("c")
    s = lax.axis_index("s")

    e0 = s * EPT
    pltpu.sync_copy(src_hbm.at[pl.ds(e0, EPT)], src_v)
    coff = c * N

    def shift(i, carry):
        src_v[pl.ds(16 * i, 16)] = src_v[pl.ds(16 * i, 16)] + coff
        return carry

    lax.fori_loop(0, EPT // 16, shift, 0)

    zero16 = jnp.zeros((16,), jnp.float32)

    def zrow(r, carry):
        for k in range(F // 16):
            rows[r, pl.ds(16 * k, 16)] = zero16
        return carry

    lax.fori_loop(0, K, zrow, 0)
    r0 = s * ROWS_PT
    iota16 = lax.iota(jnp.int32, 16)
    for q in range(ROWS_PT // K):
        for k in range(K // 16):
            idx_z[pl.ds(16 * k, 16)] = iota16 + (r0 + q * K + 16 * k)
        pltpu.sync_copy(rows.at[pl.ds(0, K)], acc.at[idx_z])
    plsc.subcore_barrier()

    R = 2

    def issue_gather(j, b):
        pltpu.async_copy(tbl_hbm.at[src_v.at[pl.ds(j * K, K)]],
                         rows.at[pl.ds(b * K, K)], gsems.at[b])

    def wait_gather(b):
        pltpu.make_async_copy(tbl_hbm.at[src_v.at[pl.ds(0, K)]],
                              rows.at[pl.ds(b * K, K)], gsems.at[b]).wait()

    def issue_scatter(b):
        pltpu.async_copy(rows.at[pl.ds(b * K, K)], acc.at[dsts.at[b]],
                         ssems.at[b], add=True)

    def wait_scatter(b):
        pltpu.make_async_copy(rows.at[pl.ds(b * K, K)], acc.at[dsts.at[b]],
                              ssems.at[b]).wait()

    def load_dst(j, b):
        pltpu.async_copy(dst_hbm.at[pl.ds(e0 + j * K, K)], dsts.at[b],
                         dsems.at[b])

    def wait_dst(b):
        pltpu.make_async_copy(dst_hbm.at[pl.ds(e0, K)], dsts.at[b],
                              dsems.at[b]).wait()

    def body(i, carry):
        @pl.when(i < NCHUNK)
        def _():
            b = lax.rem(i, R)

            @pl.when(i >= R)
            def _():
                wait_scatter(b)

            load_dst(i, b)
            issue_gather(i, b)

        @pl.when(i >= 1)
        def _():
            b2 = lax.rem(i - 1, R)
            wait_gather(b2)
            wait_dst(b2)
            issue_scatter(b2)

        return carry

    lax.fori_loop(0, NCHUNK + 1, body, 0)

    def drain(b, carry):
        wait_scatter(b)
        return carry

    lax.fori_loop(0, R, drain, 0)
    plsc.subcore_barrier()

    ob = c * NP + r0
    for q in range(ROWS_PT // K):
        pltpu.sync_copy(acc.at[pl.ds(r0 + q * K, K)], rows.at[pl.ds(0, K)])
        pltpu.sync_copy(rows.at[pl.ds(0, K)], out.at[pl.ds(ob + q * K, K)])


_RB = 1000


def _cell_body(ax_ref, ah_ref, deg_ref, h_ref, w_ref, b_ref, out_ref):
    di = 1.0 / jnp.maximum(deg_ref[...], 1.0)
    m = jnp.concatenate([ax_ref[...] * di, ah_ref[...] * di], axis=1)
    pre = jnp.dot(m, w_ref[...], preferred_element_type=jnp.float32) + b_ref[...]
    u = jax.nn.sigmoid(pre[:, :F])
    cand = jnp.tanh(pre[:, F:])
    h = h_ref[...]
    out_ref[...] = u * h + (1.0 - u) * cand


_cell = pl.pallas_call(
    _cell_body,
    grid=(N // _RB,),
    in_specs=[
        pl.BlockSpec((_RB, F), lambda i: (i, 0)),
        pl.BlockSpec((_RB, F), lambda i: (i, 0)),
        pl.BlockSpec((_RB, 1), lambda i: (i, 0)),
        pl.BlockSpec((_RB, F), lambda i: (i, 0)),
        pl.BlockSpec((2 * F, 2 * F), lambda i: (0, 0)),
        pl.BlockSpec((1, 2 * F), lambda i: (0, 0)),
    ],
    out_specs=pl.BlockSpec((_RB, F), lambda i: (i, 0)),
    out_shape=jax.ShapeDtypeStruct((N, F), jnp.float32),
)


def _proj_body(y_ref, w_ref, b_ref, out_ref):
    out_ref[...] = (
        jnp.dot(y_ref[...], w_ref[...], preferred_element_type=jnp.float32)
        + b_ref[...])


_proj = pl.pallas_call(
    _proj_body,
    grid=(NS_STEPS * N // _RB,),
    in_specs=[
        pl.BlockSpec((_RB, F), lambda i: (i, 0)),
        pl.BlockSpec((F, F), lambda i: (0, 0)),
        pl.BlockSpec((1, F), lambda i: (0, 0)),
    ],
    out_specs=pl.BlockSpec((_RB, F), lambda i: (i, 0)),
    out_shape=jax.ShapeDtypeStruct((NS_STEPS * N, F), jnp.float32),
)


def kernel(x, hidden_states, W, B, Wo, bo, edge_index):
    src = edge_index[0]
    dst = edge_index[1]

    ones_tbl = jnp.ones((2 * N, F), jnp.float32)
    deg = _agg(ones_tbl, src, dst)[:N, :1]

    wcat, bcat = [], []
    for j in range(NL):
        wj = jnp.concatenate([
            jnp.concatenate([W[j, 2], W[j, 4]], axis=1),
            jnp.concatenate([W[j, 3], W[j, 5]], axis=1)], axis=0)
        bj = jnp.concatenate([B[j, 2] + B[j, 3], B[j, 4] + B[j, 5]])[None, :]
        wcat.append(wj)
        bcat.append(bj)

    hid = [hidden_states[j] for j in range(NL)]
    outs = []
    for i in range(NS_STEPS):
        inp = x[i]
        for j in range(NL):
            tbl = jnp.concatenate([inp, hid[j]], axis=0)
            agg = _agg(tbl, src, dst)
            inp = _cell(agg[:N], agg[NP:NP + N], deg, hid[j],
                        wcat[j], bcat[j])
            hid[j] = inp
        outs.append(inp)

    y = jnp.stack(outs).reshape(NS_STEPS * N, F)
    out = _proj(y, Wo, bo[None, :]).reshape(NS_STEPS, N, F)
    return (out, jnp.stack(hid))


# speedup vs baseline: 1.2306x; 1.0192x over previous
import functools

import jax
import jax.numpy as jnp
from jax import lax
from jax.experimental import pallas as pl
from jax.experimental.pallas import tpu as pltpu
from jax.experimental.pallas import tpu_sc as plsc

N = 10000
E = 320000
F = 128
NL = 2
NS_STEPS = 4

NC = 2
NSUB = 16
K = 80
EPT = E // NSUB
NCHUNK = EPT // K
NP = 10240
ROWS_PT = NP // NSUB

_mesh = plsc.VectorSubcoreMesh(
    core_axis_name="c", subcore_axis_name="s", num_cores=NC, num_subcores=NSUB)


@functools.partial(
    pl.kernel,
    out_type=pltpu.HBM((2 * N, F), jnp.float32),
    mesh=_mesh,
    scratch_types=[
        pltpu.VMEM((EPT,), jnp.int32),
        pltpu.VMEM((2, K), jnp.int32),
        pltpu.VMEM((2 * K, F), jnp.float32),
        pltpu.VMEM((K,), jnp.int32),
        pltpu.VMEM_SHARED((NP, F), jnp.float32),
        pltpu.SemaphoreType.DMA((2,)),
        pltpu.SemaphoreType.DMA((2,)),
        pltpu.SemaphoreType.DMA((2,)),
    ],
)
def _agg(tbl_hbm, src_hbm, dst_hbm, out,
         src_v, dsts, rows, idx_z, acc, gsems, ssems, dsems):
    c = lax.axis_index("c")
    s = lax.axis_index("s")

    e0 = s * EPT
    pltpu.sync_copy(src_hbm.at[pl.ds(e0, EPT)], src_v)
    coff = c * N

    def shift(i, carry):
        src_v[pl.ds(16 * i, 16)] = src_v[pl.ds(16 * i, 16)] + coff
        return carry

    lax.fori_loop(0, EPT // 16, shift, 0)

    zero16 = jnp.zeros((16,), jnp.float32)

    def zrow(r, carry):
        for k in range(F // 16):
            rows[r, pl.ds(16 * k, 16)] = zero16
        return carry

    lax.fori_loop(0, K, zrow, 0)
    r0 = s * ROWS_PT
    iota16 = lax.iota(jnp.int32, 16)
    for q in range(ROWS_PT // K):
        for k in range(K // 16):
            idx_z[pl.ds(16 * k, 16)] = iota16 + (r0 + q * K + 16 * k)
        pltpu.sync_copy(rows.at[pl.ds(0, K)], acc.at[idx_z])
    plsc.subcore_barrier()

    R = 2

    def issue_gather(j, b):
        pltpu.async_copy(tbl_hbm.at[src_v.at[pl.ds(j * K, K)]],
                         rows.at[pl.ds(b * K, K)], gsems.at[b])

    def wait_gather(b):
        pltpu.make_async_copy(tbl_hbm.at[src_v.at[pl.ds(0, K)]],
                              rows.at[pl.ds(b * K, K)], gsems.at[b]).wait()

    def issue_scatter(b):
        pltpu.async_copy(rows.at[pl.ds(b * K, K)], acc.at[dsts.at[b]],
                         ssems.at[b], add=True)

    def wait_scatter(b):
        pltpu.make_async_copy(rows.at[pl.ds(b * K, K)], acc.at[dsts.at[b]],
                              ssems.at[b]).wait()

    def load_dst(j, b):
        pltpu.async_copy(dst_hbm.at[pl.ds(e0 + j * K, K)], dsts.at[b],
                         dsems.at[b])

    def wait_dst(b):
        pltpu.make_async_copy(dst_hbm.at[pl.ds(e0, K)], dsts.at[b],
                              dsems.at[b]).wait()

    def body(i, carry):
        @pl.when(i < NCHUNK)
        def _():
            b = lax.rem(i, R)

            @pl.when(i >= R)
            def _():
                wait_scatter(b)

            load_dst(i, b)
            issue_gather(i, b)

        @pl.when(i >= 1)
        def _():
            b2 = lax.rem(i - 1, R)
            wait_gather(b2)
            wait_dst(b2)
            issue_scatter(b2)

        return carry

    lax.fori_loop(0, NCHUNK + 1, body, 0)

    def drain(b, carry):
        wait_scatter(b)
        return carry

    lax.fori_loop(0, R, drain, 0)
    plsc.subcore_barrier()

    ob = c * N + r0
    for q in range(ROWS_PT // K):
        @pl.when(r0 + q * K < N)
        def _():
            pltpu.sync_copy(acc.at[pl.ds(r0 + q * K, K)], rows.at[pl.ds(0, K)])
            pltpu.sync_copy(rows.at[pl.ds(0, K)], out.at[pl.ds(ob + q * K, K)])


_RB = 1000


def _cell_body(ax_ref, ah_ref, deg_ref, h_ref, w_ref, b_ref, out_ref):
    di = 1.0 / jnp.maximum(deg_ref[:, :1], 1.0)
    m = jnp.concatenate([ax_ref[...] * di, ah_ref[...] * di], axis=1)
    pre = jnp.dot(m, w_ref[...], preferred_element_type=jnp.float32) + b_ref[...]
    u = jax.nn.sigmoid(pre[:, :F])
    cand = jnp.tanh(pre[:, F:])
    h = h_ref[...]
    out_ref[...] = u * h + (1.0 - u) * cand


_cell = pl.pallas_call(
    _cell_body,
    grid=(N // _RB,),
    in_specs=[
        pl.BlockSpec((_RB, F), lambda i: (i, 0)),
        pl.BlockSpec((_RB, F), lambda i: (i + N // _RB, 0)),
        pl.BlockSpec((_RB, F), lambda i: (i, 0)),
        pl.BlockSpec((_RB, F), lambda i: (i, 0)),
        pl.BlockSpec((2 * F, 2 * F), lambda i: (0, 0)),
        pl.BlockSpec((1, 2 * F), lambda i: (0, 0)),
    ],
    out_specs=pl.BlockSpec((_RB, F), lambda i: (i, 0)),
    out_shape=jax.ShapeDtypeStruct((N, F), jnp.float32),
)


def _proj_body(y_ref, w_ref, b_ref, out_ref):
    out_ref[...] = (
        jnp.dot(y_ref[...], w_ref[...], preferred_element_type=jnp.float32)
        + b_ref[...])


_proj = pl.pallas_call(
    _proj_body,
    grid=(NS_STEPS * N // _RB,),
    in_specs=[
        pl.BlockSpec((_RB, F), lambda i: (i, 0)),
        pl.BlockSpec((F, F), lambda i: (0, 0)),
        pl.BlockSpec((1, F), lambda i: (0, 0)),
    ],
    out_specs=pl.BlockSpec((_RB, F), lambda i: (i, 0)),
    out_shape=jax.ShapeDtypeStruct((NS_STEPS * N, F), jnp.float32),
)


def kernel(x, hidden_states, W, B, Wo, bo, edge_index):
    src = edge_index[0]
    dst = edge_index[1]

    ones_tbl = jnp.ones((2 * N, F), jnp.float32)
    deg = _agg(ones_tbl, src, dst)

    wcat, bcat = [], []
    for j in range(NL):
        wj = jnp.concatenate([
            jnp.concatenate([W[j, 2], W[j, 4]], axis=1),
            jnp.concatenate([W[j, 3], W[j, 5]], axis=1)], axis=0)
        bj = jnp.concatenate([B[j, 2] + B[j, 3], B[j, 4] + B[j, 5]])[None, :]
        wcat.append(wj)
        bcat.append(bj)

    hid = [hidden_states[j] for j in range(NL)]
    outs = []
    for i in range(NS_STEPS):
        inp = x[i]
        for j in range(NL):
            tbl = jnp.concatenate([inp, hid[j]], axis=0)
            agg = _agg(tbl, src, dst)
            inp = _cell(agg, agg, deg, hid[j], wcat[j], bcat[j])
            hid[j] = inp
        outs.append(inp)

    y = jnp.stack(outs).reshape(NS_STEPS * N, F)
    out = _proj(y, Wo, bo[None, :]).reshape(NS_STEPS, N, F)
    return (out, jnp.stack(hid))


# speedup vs baseline: 1.3184x; 1.0714x over previous
import functools

import jax
import jax.numpy as jnp
from jax import lax
from jax.experimental import pallas as pl
from jax.experimental.pallas import tpu as pltpu
from jax.experimental.pallas import tpu_sc as plsc

N = 10000
E = 320000
F = 128
NL = 2
NS_STEPS = 4

NC = 2
NSUB = 16
K = 80
EPT = E // NSUB
NCHUNK = EPT // K
NP = 10240
ROWS_PT = NP // NSUB

_mesh = plsc.VectorSubcoreMesh(
    core_axis_name="c", subcore_axis_name="s", num_cores=NC, num_subcores=NSUB)


@functools.partial(
    pl.kernel,
    out_type=pltpu.HBM((2 * N, F), jnp.float32),
    mesh=_mesh,
    scratch_types=[
        pltpu.VMEM((EPT,), jnp.int32),
        pltpu.VMEM((2, K), jnp.int32),
        pltpu.VMEM((2 * K, F), jnp.float32),
        pltpu.VMEM((K,), jnp.int32),
        pltpu.VMEM_SHARED((NP, F), jnp.float32),
        pltpu.SemaphoreType.DMA((2,)),
        pltpu.SemaphoreType.DMA((2,)),
        pltpu.SemaphoreType.DMA((2,)),
    ],
)
def _agg(tbl_hbm, src_hbm, dst_hbm, out,
         src_v, dsts, rows, idx_z, acc, gsems, ssems, dsems):
    c = lax.axis_index("c")
    s = lax.axis_index("s")

    e0 = s * EPT
    pltpu.sync_copy(src_hbm.at[pl.ds(e0, EPT)], src_v)
    coff = c * N

    def shift(i, carry):
        src_v[pl.ds(16 * i, 16)] = src_v[pl.ds(16 * i, 16)] + coff
        return carry

    lax.fori_loop(0, EPT // 16, shift, 0)

    zero16 = jnp.zeros((16,), jnp.float32)

    def zrow(r, carry):
        for k in range(F // 16):
            rows[r, pl.ds(16 * k, 16)] = zero16
        return carry

    lax.fori_loop(0, K, zrow, 0)
    r0 = s * ROWS_PT
    iota16 = lax.iota(jnp.int32, 16)
    for q in range(ROWS_PT // K):
        for k in range(K // 16):
            idx_z[pl.ds(16 * k, 16)] = iota16 + (r0 + q * K + 16 * k)
        pltpu.sync_copy(rows.at[pl.ds(0, K)], acc.at[idx_z])
    plsc.subcore_barrier()

    R = 2

    def issue_gather(j, b):
        pltpu.async_copy(tbl_hbm.at[src_v.at[pl.ds(j * K, K)]],
                         rows.at[pl.ds(b * K, K)], gsems.at[b])

    def wait_gather(b):
        pltpu.make_async_copy(tbl_hbm.at[src_v.at[pl.ds(0, K)]],
                              rows.at[pl.ds(b * K, K)], gsems.at[b]).wait()

    def issue_scatter(b):
        pltpu.async_copy(rows.at[pl.ds(b * K, K)], acc.at[dsts.at[b]],
                         ssems.at[b], add=True)

    def wait_scatter(b):
        pltpu.make_async_copy(rows.at[pl.ds(b * K, K)], acc.at[dsts.at[b]],
                              ssems.at[b]).wait()

    def load_dst(j, b):
        pltpu.async_copy(dst_hbm.at[pl.ds(e0 + j * K, K)], dsts.at[b],
                         dsems.at[b])

    def wait_dst(b):
        pltpu.make_async_copy(dst_hbm.at[pl.ds(e0, K)], dsts.at[b],
                              dsems.at[b]).wait()

    def body(i, carry):
        @pl.when(i < NCHUNK)
        def _():
            b = lax.rem(i, R)

            @pl.when(i >= R)
            def _():
                wait_scatter(b)

            load_dst(i, b)
            issue_gather(i, b)

        @pl.when(i >= 1)
        def _():
            b2 = lax.rem(i - 1, R)
            wait_gather(b2)
            wait_dst(b2)
            issue_scatter(b2)

        return carry

    lax.fori_loop(0, NCHUNK + 1, body, 0)

    def drain(b, carry):
        wait_scatter(b)
        return carry

    lax.fori_loop(0, R, drain, 0)
    plsc.subcore_barrier()

    ob = c * N + r0
    for q in range(ROWS_PT // K):
        @pl.when(r0 + q * K < N)
        def _():
            pltpu.sync_copy(acc.at[pl.ds(r0 + q * K, K)], rows.at[pl.ds(0, K)])
            pltpu.sync_copy(rows.at[pl.ds(0, K)], out.at[pl.ds(ob + q * K, K)])


EPT2 = E // (2 * NSUB)
NCHUNK2 = EPT2 // K


@functools.partial(
    pl.kernel,
    out_type=pltpu.HBM((2 * N, F), jnp.float32),
    mesh=_mesh,
    scratch_types=[
        pltpu.VMEM((2, K), jnp.int32),
        pltpu.VMEM((K, F), jnp.float32),
        pltpu.VMEM((K,), jnp.int32),
        pltpu.VMEM_SHARED((NP, F), jnp.float32),
        pltpu.SemaphoreType.DMA((2,)),
        pltpu.SemaphoreType.DMA((2,)),
    ],
)
def _degk(dst_hbm, out, dsts, rows, idx_z, acc, ssems, dsems):
    c = lax.axis_index("c")
    s = lax.axis_index("s")
    e0 = c * (E // 2) + s * EPT2
    zero16 = jnp.zeros((16,), jnp.float32)

    def zrow(r, carry):
        for k in range(F // 16):
            rows[r, pl.ds(16 * k, 16)] = zero16
        return carry

    lax.fori_loop(0, K, zrow, 0)
    r0 = s * ROWS_PT
    iota16 = lax.iota(jnp.int32, 16)
    for q in range(ROWS_PT // K):
        for k in range(K // 16):
            idx_z[pl.ds(16 * k, 16)] = iota16 + (r0 + q * K + 16 * k)
        pltpu.sync_copy(rows.at[pl.ds(0, K)], acc.at[idx_z])

    one16 = jnp.full((16,), 1.0, jnp.float32)

    def orow(r, carry):
        for k in range(F // 16):
            rows[r, pl.ds(16 * k, 16)] = one16
        return carry

    lax.fori_loop(0, K, orow, 0)
    plsc.subcore_barrier()

    def issue_scatter(b):
        pltpu.async_copy(rows.at[pl.ds(0, K)], acc.at[dsts.at[b]],
                         ssems.at[b], add=True)

    def wait_scatter(b):
        pltpu.make_async_copy(rows.at[pl.ds(0, K)], acc.at[dsts.at[b]],
                              ssems.at[b]).wait()

    def load_dst(j, b):
        pltpu.async_copy(dst_hbm.at[pl.ds(e0 + j * K, K)], dsts.at[b],
                         dsems.at[b])

    def wait_dst(b):
        pltpu.make_async_copy(dst_hbm.at[pl.ds(e0, K)], dsts.at[b],
                              dsems.at[b]).wait()

    def body(i, carry):
        @pl.when(i < NCHUNK2)
        def _():
            b = lax.rem(i, 2)

            @pl.when(i >= 2)
            def _():
                wait_scatter(b)

            load_dst(i, b)

        @pl.when(i >= 1)
        def _():
            b2 = lax.rem(i - 1, 2)
            wait_dst(b2)
            issue_scatter(b2)

        return carry

    lax.fori_loop(0, NCHUNK2 + 1, body, 0)

    def drain(b, carry):
        wait_scatter(b)
        return carry

    lax.fori_loop(0, 2, drain, 0)
    plsc.subcore_barrier()

    ob = c * N + r0
    for q in range(ROWS_PT // K):
        @pl.when(r0 + q * K < N)
        def _():
            pltpu.sync_copy(acc.at[pl.ds(r0 + q * K, K)], rows.at[pl.ds(0, K)])
            pltpu.sync_copy(rows.at[pl.ds(0, K)], out.at[pl.ds(ob + q * K, K)])


_RB = 1000


def _cell_body(ax_ref, ah_ref, d0_ref, d1_ref, h_ref, w_ref, b_ref, out_ref):
    di = 1.0 / jnp.maximum(d0_ref[:, :1] + d1_ref[:, :1], 1.0)
    m = jnp.concatenate([ax_ref[...] * di, ah_ref[...] * di], axis=1)
    pre = jnp.dot(m, w_ref[...], preferred_element_type=jnp.float32) + b_ref[...]
    u = jax.nn.sigmoid(pre[:, :F])
    cand = jnp.tanh(pre[:, F:])
    h = h_ref[...]
    out_ref[...] = u * h + (1.0 - u) * cand


_cell = pl.pallas_call(
    _cell_body,
    grid=(N // _RB,),
    in_specs=[
        pl.BlockSpec((_RB, F), lambda i: (i, 0)),
        pl.BlockSpec((_RB, F), lambda i: (i + N // _RB, 0)),
        pl.BlockSpec((_RB, F), lambda i: (i, 0)),
        pl.BlockSpec((_RB, F), lambda i: (i + N // _RB, 0)),
        pl.BlockSpec((_RB, F), lambda i: (i, 0)),
        pl.BlockSpec((2 * F, 2 * F), lambda i: (0, 0)),
        pl.BlockSpec((1, 2 * F), lambda i: (0, 0)),
    ],
    out_specs=pl.BlockSpec((_RB, F), lambda i: (i, 0)),
    out_shape=jax.ShapeDtypeStruct((N, F), jnp.float32),
)


def _proj_body(y_ref, w_ref, b_ref, out_ref):
    out_ref[...] = (
        jnp.dot(y_ref[...], w_ref[...], preferred_element_type=jnp.float32)
        + b_ref[...])


_proj = pl.pallas_call(
    _proj_body,
    grid=(NS_STEPS * N // _RB,),
    in_specs=[
        pl.BlockSpec((_RB, F), lambda i: (i, 0)),
        pl.BlockSpec((F, F), lambda i: (0, 0)),
        pl.BlockSpec((1, F), lambda i: (0, 0)),
    ],
    out_specs=pl.BlockSpec((_RB, F), lambda i: (i, 0)),
    out_shape=jax.ShapeDtypeStruct((NS_STEPS * N, F), jnp.float32),
)


def kernel(x, hidden_states, W, B, Wo, bo, edge_index):
    src = edge_index[0]
    dst = edge_index[1]

    deg = _degk(dst)

    wcat, bcat = [], []
    for j in range(NL):
        wj = jnp.concatenate([
            jnp.concatenate([W[j, 2], W[j, 4]], axis=1),
            jnp.concatenate([W[j, 3], W[j, 5]], axis=1)], axis=0)
        bj = jnp.concatenate([B[j, 2] + B[j, 3], B[j, 4] + B[j, 5]])[None, :]
        wcat.append(wj)
        bcat.append(bj)

    hid = [hidden_states[j] for j in range(NL)]
    outs = []
    for i in range(NS_STEPS):
        inp = x[i]
        for j in range(NL):
            tbl = jnp.concatenate([inp, hid[j]], axis=0)
            agg = _agg(tbl, src, dst)
            inp = _cell(agg, agg, deg, deg, hid[j], wcat[j], bcat[j])
            hid[j] = inp
        outs.append(inp)

    y = jnp.stack(outs).reshape(NS_STEPS * N, F)
    out = _proj(y, Wo, bo[None, :]).reshape(NS_STEPS, N, F)
    return (out, jnp.stack(hid))
